# Initial kernel scaffold; baseline (speedup 1.0000x reference)
import jax
import jax.numpy as jnp
from jax.experimental import pallas as pl


def kernel(x, edge_index, W):
    raise NotImplementedError("write your pallas kernel here")


# speedup vs baseline: 5.5969x; 5.5969x over previous
import functools

import jax
import jax.numpy as jnp
from jax import lax
from jax.experimental import pallas as pl
from jax.experimental.pallas import tpu as pltpu
from jax.experimental.pallas import tpu_sc as plsc

NC = 2
NS = 16
CHUNK = 128
WSC = 80
HALF = 64


SEG = 40


def _sc_scatter_body(cpt, rpt, xcat_hbm, src_hbm, dst_hbm, zeros_hbm, out_hbm,
                     src_v, dst_v, rows_v, acc_sh, sem):
    cid = lax.axis_index("c")
    sid = lax.axis_index("s")

    pltpu.sync_copy(zeros_hbm, acc_sh.at[pl.ds(sid * rpt, rpt)])
    plsc.subcore_barrier()

    def seg_body(s, carry):
        pltpu.sync_copy(
            src_hbm.at[pl.ds((cid * NS + sid) * cpt + s * SEG, SEG)], src_v)
        pltpu.sync_copy(dst_hbm.at[pl.ds(sid * cpt + s * SEG, SEG)], dst_v)

        def body(j, c2):
            pltpu.async_copy(xcat_hbm.at[src_v.at[j]], rows_v, sem).wait()
            pltpu.sync_copy(rows_v, acc_sh.at[dst_v.at[j]], add=True)
            return c2

        lax.fori_loop(0, SEG, body, 0)
        return carry

    lax.fori_loop(0, cpt // SEG, seg_body, 0)
    plsc.subcore_barrier()
    pltpu.sync_copy(acc_sh.at[pl.ds(sid * rpt, rpt)],
                    out_hbm.at[cid, pl.ds(sid * rpt, rpt)])


def _tc_matmul_body(h, acc_ref, x_ref, w_ref, o_ref):
    a0 = acc_ref[0]
    a1 = acc_ref[1]
    g_lo = a0[:, :HALF]
    deg = a0[:, HALF:HALF + 1]
    g_hi = a1[:, :HALF]
    o_ref[...] = (
        jnp.dot(g_lo, w_ref[:HALF], preferred_element_type=jnp.float32)
        + jnp.dot(g_hi, w_ref[HALF:h], preferred_element_type=jnp.float32)
        + jnp.dot(deg * x_ref[...], w_ref[h:], preferred_element_type=jnp.float32)
    )


def kernel(x, edge_index, W):
    n, h = x.shape
    e = edge_index.shape[1]

    rpt = (-(-(n + 1) // NS) + 7) // 8 * 8
    n_acc = rpt * NS

    cpt = -(-(-(-(2 * e) // (NS * CHUNK))) // SEG) * SEG
    pad_e = NS * cpt * CHUNK - 2 * e
    src = edge_index[0].astype(jnp.int32)
    dst = edge_index[1].astype(jnp.int32)
    pad = jnp.full((pad_e,), n, jnp.int32)
    src_all = jnp.concatenate([src, dst, pad]).reshape(NS * cpt, CHUNK)
    dst_all = jnp.concatenate([dst, src, pad]).reshape(NS * cpt, CHUNK)
    src_both = jnp.concatenate([src_all, src_all + n_acc], axis=0)

    xcat = (jnp.zeros((2 * n_acc, WSC), jnp.float32)
            .at[:n, :HALF].set(x[:, :HALF])
            .at[:n, HALF].set(1.0)
            .at[n_acc:n_acc + n, :HALF].set(x[:, HALF:]))
    zeros_blk = jnp.zeros((rpt, WSC), jnp.float32)

    sc_scatter = pl.kernel(
        functools.partial(_sc_scatter_body, cpt, rpt),
        out_type=jax.ShapeDtypeStruct((NC, n_acc, WSC), jnp.float32),
        mesh=plsc.VectorSubcoreMesh(core_axis_name="c", subcore_axis_name="s",
                                    num_cores=NC, num_subcores=NS),
        scratch_types=[
            pltpu.VMEM((SEG, CHUNK), jnp.int32),
            pltpu.VMEM((SEG, CHUNK), jnp.int32),
            pltpu.VMEM((CHUNK, WSC), jnp.float32),
            pltpu.VMEM_SHARED((n_acc, WSC), jnp.float32),
            pltpu.SemaphoreType.DMA,
        ],
        compiler_params=pltpu.CompilerParams(use_tc_tiling_on_sc=False),
    )
    acc = sc_scatter(xcat, src_both, dst_all, zeros_blk)

    x_pad = jnp.zeros((n_acc, h), jnp.float32).at[:n].set(x)
    out = pl.pallas_call(
        functools.partial(_tc_matmul_body, h),
        grid=(NS,),
        in_specs=[
            pl.BlockSpec((NC, rpt, WSC), lambda i: (0, i, 0)),
            pl.BlockSpec((rpt, h), lambda i: (i, 0)),
            pl.BlockSpec((2 * h, h), lambda i: (0, 0)),
        ],
        out_specs=pl.BlockSpec((rpt, h), lambda i: (i, 0)),
        out_shape=jax.ShapeDtypeStruct((n_acc, h), jnp.float32),
    )(acc, x_pad, W)
    return out[:n]


# speedup vs baseline: 6.2355x; 1.1141x over previous
import functools

import jax
import jax.numpy as jnp
from jax import lax
from jax.experimental import pallas as pl
from jax.experimental.pallas import tpu as pltpu
from jax.experimental.pallas import tpu_sc as plsc

NC = 2
NS = 16
CHUNK = 128
WSC = 80
HALF = 64


SEG = 40


def _sc_scatter_body(cpt, rpt, xcat_hbm, src_hbm, dst_hbm, zeros_hbm, out_hbm,
                     src_v, dst_v, rows_v, acc_sh, sem_g0, sem_g1, sem_s0,
                     sem_s1):
    cid = lax.axis_index("c")
    sid = lax.axis_index("s")
    sem_g = (sem_g0, sem_g1)
    sem_s = (sem_s0, sem_s1)

    def start_gather(j, b):
        pltpu.async_copy(xcat_hbm.at[src_v.at[j]], rows_v.at[b], sem_g[b])

    def wait_gather(j, b):
        pltpu.make_async_copy(xcat_hbm.at[src_v.at[j]], rows_v.at[b],
                              sem_g[b]).wait()

    def start_scatter(j, b):
        pltpu.async_copy(rows_v.at[b], acc_sh.at[dst_v.at[j]], sem_s[b],
                         add=True)

    def wait_scatter(j, b):
        pltpu.make_async_copy(rows_v.at[b], acc_sh.at[dst_v.at[j]],
                              sem_s[b]).wait()

    pltpu.sync_copy(zeros_hbm, acc_sh.at[pl.ds(sid * rpt, rpt)])
    plsc.subcore_barrier()

    def seg_body(s, carry):
        pltpu.sync_copy(
            src_hbm.at[pl.ds((cid * NS + sid) * cpt + s * SEG, SEG)], src_v)
        pltpu.sync_copy(dst_hbm.at[pl.ds(sid * cpt + s * SEG, SEG)], dst_v)

        start_gather(0, 0)

        def body(j0, c2):
            for b in range(2):
                j = 2 * j0 + b
                wait_gather(j, b)

                @pl.when(j > 0)
                def _():
                    wait_scatter(j - 1, 1 - b)

                @pl.when(j < SEG - 1)
                def _():
                    start_gather(j + 1, 1 - b)

                start_scatter(j, b)
            return c2

        lax.fori_loop(0, SEG // 2, body, 0)
        wait_scatter(SEG - 1, 1)
        return carry

    lax.fori_loop(0, cpt // SEG, seg_body, 0)
    plsc.subcore_barrier()
    pltpu.sync_copy(acc_sh.at[pl.ds(sid * rpt, rpt)],
                    out_hbm.at[cid, pl.ds(sid * rpt, rpt)])


def _tc_matmul_body(h, acc_ref, x_ref, w_ref, o_ref):
    a0 = acc_ref[0]
    a1 = acc_ref[1]
    g_lo = a0[:, :HALF]
    deg = a0[:, HALF:HALF + 1]
    g_hi = a1[:, :HALF]
    o_ref[...] = (
        jnp.dot(g_lo, w_ref[:HALF], preferred_element_type=jnp.float32)
        + jnp.dot(g_hi, w_ref[HALF:h], preferred_element_type=jnp.float32)
        + jnp.dot(deg * x_ref[...], w_ref[h:], preferred_element_type=jnp.float32)
    )


def kernel(x, edge_index, W):
    n, h = x.shape
    e = edge_index.shape[1]

    rpt = (-(-(n + 1) // NS) + 7) // 8 * 8
    n_acc = rpt * NS

    cpt = -(-(-(-(2 * e) // (NS * CHUNK))) // SEG) * SEG
    pad_e = NS * cpt * CHUNK - 2 * e
    src = edge_index[0].astype(jnp.int32)
    dst = edge_index[1].astype(jnp.int32)
    pad = jnp.full((pad_e,), n, jnp.int32)
    src_all = jnp.concatenate([src, dst, pad]).reshape(NS * cpt, CHUNK)
    dst_all = jnp.concatenate([dst, src, pad]).reshape(NS * cpt, CHUNK)
    src_both = jnp.concatenate([src_all, src_all + n_acc], axis=0)

    xcat = (jnp.zeros((2 * n_acc, WSC), jnp.float32)
            .at[:n, :HALF].set(x[:, :HALF])
            .at[:n, HALF].set(1.0)
            .at[n_acc:n_acc + n, :HALF].set(x[:, HALF:]))
    zeros_blk = jnp.zeros((rpt, WSC), jnp.float32)

    sc_scatter = pl.kernel(
        functools.partial(_sc_scatter_body, cpt, rpt),
        out_type=jax.ShapeDtypeStruct((NC, n_acc, WSC), jnp.float32),
        mesh=plsc.VectorSubcoreMesh(core_axis_name="c", subcore_axis_name="s",
                                    num_cores=NC, num_subcores=NS),
        scratch_types=[
            pltpu.VMEM((SEG, CHUNK), jnp.int32),
            pltpu.VMEM((SEG, CHUNK), jnp.int32),
            pltpu.VMEM((2, CHUNK, WSC), jnp.float32),
            pltpu.VMEM_SHARED((n_acc, WSC), jnp.float32),
            pltpu.SemaphoreType.DMA,
            pltpu.SemaphoreType.DMA,
            pltpu.SemaphoreType.DMA,
            pltpu.SemaphoreType.DMA,
        ],
        compiler_params=pltpu.CompilerParams(use_tc_tiling_on_sc=False),
    )
    acc = sc_scatter(xcat, src_both, dst_all, zeros_blk)

    x_pad = jnp.zeros((n_acc, h), jnp.float32).at[:n].set(x)
    out = pl.pallas_call(
        functools.partial(_tc_matmul_body, h),
        grid=(NS,),
        in_specs=[
            pl.BlockSpec((NC, rpt, WSC), lambda i: (0, i, 0)),
            pl.BlockSpec((rpt, h), lambda i: (i, 0)),
            pl.BlockSpec((2 * h, h), lambda i: (0, 0)),
        ],
        out_specs=pl.BlockSpec((rpt, h), lambda i: (i, 0)),
        out_shape=jax.ShapeDtypeStruct((n_acc, h), jnp.float32),
    )(acc, x_pad, W)
    return out[:n]


# speedup vs baseline: 7.3063x; 1.1717x over previous
import functools

import jax
import jax.numpy as jnp
from jax import lax
from jax.experimental import pallas as pl
from jax.experimental.pallas import tpu as pltpu
from jax.experimental.pallas import tpu_sc as plsc

NC = 2
NS = 16
CHUNK = 128
WSC = 80
HALF = 64


SEG = 40


NBUF = 4


def _sc_scatter_body(cpt, rpt, xcat_hbm, src_hbm, dst_hbm, zeros_hbm, out_hbm,
                     src_v, dst_v, rows_v, acc_sh, *sems):
    cid = lax.axis_index("c")
    sid = lax.axis_index("s")
    sem_g = sems[:NBUF]
    sem_s = sems[NBUF:]

    def start_gather(j, b):
        pltpu.async_copy(xcat_hbm.at[src_v.at[j]], rows_v.at[b], sem_g[b])

    def wait_gather(j, b):
        pltpu.make_async_copy(xcat_hbm.at[src_v.at[j]], rows_v.at[b],
                              sem_g[b]).wait()

    def start_scatter(j, b):
        pltpu.async_copy(rows_v.at[b], acc_sh.at[dst_v.at[j]], sem_s[b],
                         add=True)

    def wait_scatter(j, b):
        pltpu.make_async_copy(rows_v.at[b], acc_sh.at[dst_v.at[j]],
                              sem_s[b]).wait()

    pltpu.sync_copy(zeros_hbm, acc_sh.at[pl.ds(sid * rpt, rpt)])
    plsc.subcore_barrier()

    def seg_body(s, carry):
        pltpu.sync_copy(
            src_hbm.at[pl.ds((cid * NS + sid) * cpt + s * SEG, SEG)], src_v)
        pltpu.sync_copy(dst_hbm.at[pl.ds(sid * cpt + s * SEG, SEG)], dst_v)

        start_gather(0, 0)
        start_gather(1, 1)

        def body(j0, c2):
            for u in range(NBUF):
                j = NBUF * j0 + u
                b = u

                @pl.when(j >= 2)
                def _():
                    wait_scatter(j - 2, (u - 2) % NBUF)

                @pl.when(j < SEG - 2)
                def _():
                    start_gather(j + 2, (u + 2) % NBUF)

                wait_gather(j, b)
                start_scatter(j, b)
            return c2

        lax.fori_loop(0, SEG // NBUF, body, 0)
        wait_scatter(SEG - 2, (SEG - 2) % NBUF)
        wait_scatter(SEG - 1, (SEG - 1) % NBUF)
        return carry

    lax.fori_loop(0, cpt // SEG, seg_body, 0)
    plsc.subcore_barrier()
    pltpu.sync_copy(acc_sh.at[pl.ds(sid * rpt, rpt)],
                    out_hbm.at[cid, pl.ds(sid * rpt, rpt)])


def _tc_matmul_body(h, acc_ref, x_ref, w_ref, o_ref):
    a0 = acc_ref[0]
    a1 = acc_ref[1]
    g_lo = a0[:, :HALF]
    deg = a0[:, HALF:HALF + 1]
    g_hi = a1[:, :HALF]
    o_ref[...] = (
        jnp.dot(g_lo, w_ref[:HALF], preferred_element_type=jnp.float32)
        + jnp.dot(g_hi, w_ref[HALF:h], preferred_element_type=jnp.float32)
        + jnp.dot(deg * x_ref[...], w_ref[h:], preferred_element_type=jnp.float32)
    )


def kernel(x, edge_index, W):
    n, h = x.shape
    e = edge_index.shape[1]

    rpt = (-(-(n + 1) // NS) + 7) // 8 * 8
    n_acc = rpt * NS

    cpt = -(-(-(-(2 * e) // (NS * CHUNK))) // SEG) * SEG
    pad_e = NS * cpt * CHUNK - 2 * e
    src = edge_index[0].astype(jnp.int32)
    dst = edge_index[1].astype(jnp.int32)
    pad = jnp.full((pad_e,), n, jnp.int32)
    src_all = jnp.concatenate([src, dst, pad]).reshape(NS * cpt, CHUNK)
    dst_all = jnp.concatenate([dst, src, pad]).reshape(NS * cpt, CHUNK)
    src_both = jnp.concatenate([src_all, src_all + n_acc], axis=0)

    xcat = (jnp.zeros((2 * n_acc, WSC), jnp.float32)
            .at[:n, :HALF].set(x[:, :HALF])
            .at[:n, HALF].set(1.0)
            .at[n_acc:n_acc + n, :HALF].set(x[:, HALF:]))
    zeros_blk = jnp.zeros((rpt, WSC), jnp.float32)

    sc_scatter = pl.kernel(
        functools.partial(_sc_scatter_body, cpt, rpt),
        out_type=jax.ShapeDtypeStruct((NC, n_acc, WSC), jnp.float32),
        mesh=plsc.VectorSubcoreMesh(core_axis_name="c", subcore_axis_name="s",
                                    num_cores=NC, num_subcores=NS),
        scratch_types=[
            pltpu.VMEM((SEG, CHUNK), jnp.int32),
            pltpu.VMEM((SEG, CHUNK), jnp.int32),
            pltpu.VMEM((NBUF, CHUNK, WSC), jnp.float32),
            pltpu.VMEM_SHARED((n_acc, WSC), jnp.float32),
        ] + [pltpu.SemaphoreType.DMA] * (2 * NBUF),
        compiler_params=pltpu.CompilerParams(use_tc_tiling_on_sc=False),
    )
    acc = sc_scatter(xcat, src_both, dst_all, zeros_blk)

    x_pad = jnp.zeros((n_acc, h), jnp.float32).at[:n].set(x)
    out = pl.pallas_call(
        functools.partial(_tc_matmul_body, h),
        grid=(NS,),
        in_specs=[
            pl.BlockSpec((NC, rpt, WSC), lambda i: (0, i, 0)),
            pl.BlockSpec((rpt, h), lambda i: (i, 0)),
            pl.BlockSpec((2 * h, h), lambda i: (0, 0)),
        ],
        out_specs=pl.BlockSpec((rpt, h), lambda i: (i, 0)),
        out_shape=jax.ShapeDtypeStruct((n_acc, h), jnp.float32),
    )(acc, x_pad, W)
    return out[:n]
